# Initial kernel scaffold; baseline (speedup 1.0000x reference)
import jax
import jax.numpy as jnp
from jax.experimental import pallas as pl


def kernel(anchors, gt_boxes, obj_labels):
    raise NotImplementedError("write your pallas kernel here")


# speedup vs baseline: 27.2121x; 27.2121x over previous
import functools

import jax
import jax.numpy as jnp
from jax.experimental import pallas as pl
from jax.experimental.pallas import tpu as pltpu

POS_IOU_THRES = 0.7
NEG_IOU_THRES = 0.3
N_SAMPLE = 256
INPUT_H = 512.0
INPUT_W = 512.0

_N = 20000
_NP = 20480
_C = 2048
_G = 64


def _tg_kernel(a_ref, gt_ref, gmat_ref, boxes_ref, loc_ref, lab_ref, cls_ref,
               iou_scr):
    nchunks = _NP // _C

    gtc = gt_ref[0]
    gy1 = gtc[:, 0:1]
    gx1 = gtc[:, 1:2]
    gy2 = gtc[:, 2:3]
    gx2 = gtc[:, 3:4]
    area_g = jnp.maximum(gy2 - gy1, 0.0) * jnp.maximum(gx2 - gx1, 0.0)

    def anchor_chunk(c):
        s = slice(c * _C, (c + 1) * _C)
        ay1 = a_ref[0, 0:1, s]
        ax1 = a_ref[0, 1:2, s]
        ay2 = a_ref[0, 2:3, s]
        ax2 = a_ref[0, 3:4, s]
        return ay1, ax1, ay2, ax2

    def iou_chunk(ay1, ax1, ay2, ax2):
        ih = jnp.maximum(jnp.minimum(ay2, gy2) - jnp.maximum(ay1, gy1), 0.0)
        iw = jnp.maximum(jnp.minimum(ax2, gx2) - jnp.maximum(ax1, gx1), 0.0)
        inter = ih * iw
        area_a = (jnp.maximum(ay2 - ay1, 0.0)
                  * jnp.maximum(ax2 - ax1, 0.0))
        inside = ((ay1 >= 0.0) & (ax1 >= 0.0)
                  & (ay2 <= INPUT_H) & (ax2 <= INPUT_W))
        iou = inter / (area_a + area_g - inter + 1e-8)
        return jnp.where(inside, iou, 0.0), inside

    gt_best = jnp.full((_G, 1), 0.0, jnp.float32)
    for c in range(nchunks):
        iou, _ = iou_chunk(*anchor_chunk(c))
        iou_scr[:, c * _C:(c + 1) * _C] = iou
        gt_best = jnp.maximum(gt_best, jnp.max(iou, axis=1, keepdims=True))

    gmat = gmat_ref[0]
    giota = jax.lax.broadcasted_iota(jnp.int32, (_G, _C), 0)
    best_pos = gt_best > 0.0

    label_parts = []
    matched_parts = []
    for c in range(nchunks):
        ay1, ax1, ay2, ax2 = anchor_chunk(c)
        iou = iou_scr[:, c * _C:(c + 1) * _C]
        max_iou = jnp.max(iou, axis=0, keepdims=True)
        is_max = iou == max_iou
        gt_idx = jnp.min(jnp.where(is_max, giota, _G), axis=0,
                         keepdims=True)
        is_best = jnp.max(jnp.where((iou == gt_best) & best_pos, 1, 0),
                          axis=0, keepdims=True) > 0
        inside = ((ay1 >= 0.0) & (ax1 >= 0.0)
                  & (ay2 <= INPUT_H) & (ax2 <= INPUT_W))

        label = jnp.full((1, _C), -1.0, jnp.float32)
        label = jnp.where(max_iou < NEG_IOU_THRES, 0.0, label)
        label = jnp.where(is_best, 1.0, label)
        label = jnp.where(max_iou >= POS_IOU_THRES, 1.0, label)
        label = jnp.where(inside, label, -1.0)
        label_parts.append(label)

        onehot = (giota == gt_idx).astype(jnp.float32)
        gathered = jax.lax.dot_general(
            gmat, onehot, (((1,), (0,)), ((), ())),
            preferred_element_type=jnp.float32)
        by1 = gathered[0:1, :]
        bx1 = gathered[1:2, :]
        by2 = gathered[2:3, :]
        bx2 = gathered[3:4, :]
        matched_parts.append(gathered[4:5, :])

        boxes_ref[0, :, c * _C:(c + 1) * _C] = jnp.concatenate(
            [by1, bx1, by2, bx2], axis=0)

        ah = jnp.maximum(ay2 - ay1, 1e-6)
        aw = jnp.maximum(ax2 - ax1, 1e-6)
        gh = jnp.maximum(by2 - by1, 1e-6)
        gw = jnp.maximum(bx2 - bx1, 1e-6)
        ty = (by1 + 0.5 * gh - ay1 - 0.5 * ah) / ah
        tx = (bx1 + 0.5 * gw - ax1 - 0.5 * aw) / aw
        th = jnp.log(gh / ah)
        tw = jnp.log(gw / aw)
        loc_ref[0, :, c * _C:(c + 1) * _C] = jnp.concatenate(
            [ty, tx, th, tw], axis=0)

    label = jnp.concatenate(label_parts, axis=1)
    matched = jnp.concatenate(matched_parts, axis=1)

    rr = _NP // 128
    i128 = jax.lax.broadcasted_iota(jnp.int32, (128, 128), 0)
    j128 = jax.lax.broadcasted_iota(jnp.int32, (128, 128), 1)
    t128 = (i128 <= j128).astype(jnp.float32)
    ir = jax.lax.broadcasted_iota(jnp.int32, (rr, rr), 0)
    jr = jax.lax.broadcasted_iota(jnp.int32, (rr, rr), 1)
    tlow = (jr < ir).astype(jnp.float32)

    def prefix_sum(mask):
        x = mask.astype(jnp.float32).reshape(rr, 128)
        row_pref = jax.lax.dot_general(
            x, t128, (((1,), (0,)), ((), ())),
            preferred_element_type=jnp.float32)
        row_sums = jnp.sum(x, axis=1, keepdims=True)
        offsets = jax.lax.dot_general(
            tlow, row_sums, (((1,), (0,)), ((), ())),
            preferred_element_type=jnp.float32)
        return (row_pref + offsets).reshape(1, _NP)

    n_pos_target = N_SAMPLE // 2
    pos_mask = label == 1.0
    pos_rank = prefix_sum(pos_mask)
    label = jnp.where(pos_mask & (pos_rank > n_pos_target), -1.0, label)
    total_pos = jnp.sum(pos_mask.astype(jnp.float32))
    n_neg = N_SAMPLE - jnp.minimum(total_pos, float(n_pos_target))
    neg_mask = label == 0.0
    neg_rank = prefix_sum(neg_mask)
    label = jnp.where(neg_mask & (neg_rank > n_neg), -1.0, label)

    cls = jnp.where(label == 1.0, matched + 1.0,
                    jnp.where(label == 0.0, 0.0, -1.0))
    lab_ref[0] = label
    cls_ref[0] = cls.astype(jnp.int32)


@jax.jit
def kernel(anchors, gt_boxes, obj_labels):
    B = anchors.shape[0]
    a_t = jnp.transpose(anchors, (0, 2, 1))
    a_t = jnp.pad(a_t, ((0, 0), (0, 0), (0, _NP - _N)), constant_values=-1.0)
    gmat = jnp.concatenate(
        [jnp.transpose(gt_boxes, (0, 2, 1)),
         obj_labels[:, None, :].astype(jnp.float32)], axis=1)

    boxes_t, loc_t, label, cls = pl.pallas_call(
        _tg_kernel,
        grid=(B,),
        in_specs=[
            pl.BlockSpec((1, 4, _NP), lambda b: (b, 0, 0)),
            pl.BlockSpec((1, _G, 4), lambda b: (b, 0, 0)),
            pl.BlockSpec((1, 5, _G), lambda b: (b, 0, 0)),
        ],
        out_specs=[
            pl.BlockSpec((1, 4, _NP), lambda b: (b, 0, 0)),
            pl.BlockSpec((1, 4, _NP), lambda b: (b, 0, 0)),
            pl.BlockSpec((1, 1, _NP), lambda b: (b, 0, 0)),
            pl.BlockSpec((1, 1, _NP), lambda b: (b, 0, 0)),
        ],
        out_shape=[
            jax.ShapeDtypeStruct((B, 4, _NP), jnp.float32),
            jax.ShapeDtypeStruct((B, 4, _NP), jnp.float32),
            jax.ShapeDtypeStruct((B, 1, _NP), jnp.float32),
            jax.ShapeDtypeStruct((B, 1, _NP), jnp.int32),
        ],
        scratch_shapes=[pltpu.VMEM((_G, _NP), jnp.float32)],
    )(a_t, gt_boxes, gmat)

    boxes = jnp.transpose(boxes_t[:, :, :_N], (0, 2, 1))
    loc = jnp.transpose(loc_t[:, :, :_N], (0, 2, 1))
    return boxes, loc, label[:, 0, :_N], cls[:, 0, :_N]
